# Initial kernel scaffold; baseline (speedup 1.0000x reference)
import jax
import jax.numpy as jnp
from jax.experimental import pallas as pl


def kernel(x, target):
    raise NotImplementedError("write your pallas kernel here")


# speedup vs baseline: 2.3201x; 2.3201x over previous
import math

import jax
import jax.numpy as jnp
from jax import lax
from jax.experimental import pallas as pl
from jax.experimental.pallas import tpu as pltpu
from jax.experimental.pallas import tpu_sc as plsc

N = 2048
SIZE = 32000
SMOOTHING = 0.1
EPS = SMOOTHING / (SIZE - 1)
CONF = 1.0 - SMOOTHING
C0 = (SIZE - 1) * EPS * math.log(EPS) + CONF * math.log(CONF)

NC = 2
NS = 16
LANES = 16
NW = NC * NS
BPW = N // NW

ROWS_PER_BLOCK = 64
GRID = N // ROWS_PER_BLOCK


def _sc_gather_body(xflat, tgt, out, tgt_v, idx_v, val_v, sem):
    wid = lax.axis_index("s") * NC + lax.axis_index("c")
    base = wid * BPW
    pltpu.sync_copy(tgt.at[pl.ds(base, BPW)], tgt_v)
    for j in range(BPW // LANES):
        rows = (base + j * LANES) + lax.broadcasted_iota(jnp.int32, (LANES,), 0)
        idx_v[pl.ds(j * LANES, LANES)] = rows * SIZE + tgt_v[pl.ds(j * LANES, LANES)]
    pltpu.async_copy(xflat.at[idx_v], val_v, sem).wait()
    pltpu.sync_copy(val_v, out.at[pl.ds(base, BPW)])


def _sc_gather(xflat, tgt):
    k = pl.kernel(
        _sc_gather_body,
        out_type=jax.ShapeDtypeStruct((N,), jnp.float32),
        mesh=plsc.VectorSubcoreMesh(core_axis_name="c", subcore_axis_name="s"),
        scratch_types=[
            pltpu.VMEM((BPW,), jnp.int32),
            pltpu.VMEM((BPW,), jnp.int32),
            pltpu.VMEM((BPW,), jnp.float32),
            pltpu.SemaphoreType.DMA,
        ],
    )
    return k(xflat, tgt)


def _tc_loss_body(x_ref, g_ref, out_ref):
    i = pl.program_id(0)

    @pl.when(i == 0)
    def _init():
        out_ref[0, 0] = jnp.float32(0.0)

    out_ref[0, 0] += jnp.sum(x_ref[...])

    @pl.when(i == GRID - 1)
    def _fin():
        s = out_ref[0, 0]
        g = jnp.sum(g_ref[...])
        out_ref[0, 0] = (
            jnp.float32(C0)
            - jnp.float32(EPS) * (s / N)
            + jnp.float32(EPS - CONF) * (g / N)
        )


def _tc_loss(x, gvals):
    g2 = gvals.reshape(LANES, N // LANES)
    out = pl.pallas_call(
        _tc_loss_body,
        grid=(GRID,),
        in_specs=[
            pl.BlockSpec((ROWS_PER_BLOCK, SIZE), lambda i: (i, 0)),
            pl.BlockSpec((LANES, N // LANES), lambda i: (0, 0)),
        ],
        out_specs=pl.BlockSpec(memory_space=pltpu.SMEM),
        out_shape=jax.ShapeDtypeStruct((1, 1), jnp.float32),
    )(x, g2)
    return out[0, 0]


def kernel(x, target):
    tgt = target.astype(jnp.int32)
    g = _sc_gather(x.reshape(N * SIZE), tgt)
    return _tc_loss(x, g)


# speedup vs baseline: 2.3854x; 1.0282x over previous
import math

import jax
import jax.numpy as jnp
from jax import lax
from jax.experimental import pallas as pl
from jax.experimental.pallas import tpu as pltpu
from jax.experimental.pallas import tpu_sc as plsc

N = 2048
SIZE = 32000
SMOOTHING = 0.1
EPS = SMOOTHING / (SIZE - 1)
CONF = 1.0 - SMOOTHING
C0 = (SIZE - 1) * EPS * math.log(EPS) + CONF * math.log(CONF)

NC = 2
NS = 16
LANES = 16
NW = NC * NS
BPW = N // NW

ROWS_PER_BLOCK = 128
GRID = N // ROWS_PER_BLOCK


def _sc_gather_body(xflat, tgt, out, tgt_v, idx_v, val_v, sem):
    wid = lax.axis_index("s") * NC + lax.axis_index("c")
    base = wid * BPW
    pltpu.sync_copy(tgt.at[pl.ds(base, BPW)], tgt_v)
    for j in range(BPW // LANES):
        rows = (base + j * LANES) + lax.broadcasted_iota(jnp.int32, (LANES,), 0)
        idx_v[pl.ds(j * LANES, LANES)] = rows * SIZE + tgt_v[pl.ds(j * LANES, LANES)]
    pltpu.async_copy(xflat.at[idx_v], val_v, sem).wait()
    pltpu.sync_copy(val_v, out.at[pl.ds(base, BPW)])


def _sc_gather(xflat, tgt):
    k = pl.kernel(
        _sc_gather_body,
        out_type=jax.ShapeDtypeStruct((N,), jnp.float32),
        mesh=plsc.VectorSubcoreMesh(core_axis_name="c", subcore_axis_name="s"),
        scratch_types=[
            pltpu.VMEM((BPW,), jnp.int32),
            pltpu.VMEM((BPW,), jnp.int32),
            pltpu.VMEM((BPW,), jnp.float32),
            pltpu.SemaphoreType.DMA,
        ],
    )
    return k(xflat, tgt)


def _tc_loss_body(x_ref, g_ref, out_ref):
    i = pl.program_id(0)

    @pl.when(i == 0)
    def _init():
        out_ref[0, 0] = jnp.float32(0.0)

    out_ref[0, 0] += jnp.sum(x_ref[...])

    @pl.when(i == GRID - 1)
    def _fin():
        s = out_ref[0, 0]
        g = jnp.sum(g_ref[...])
        out_ref[0, 0] = (
            jnp.float32(C0)
            - jnp.float32(EPS) * (s / N)
            + jnp.float32(EPS - CONF) * (g / N)
        )


def _tc_loss(x, gvals):
    g2 = gvals.reshape(LANES, N // LANES)
    out = pl.pallas_call(
        _tc_loss_body,
        grid=(GRID,),
        in_specs=[
            pl.BlockSpec((ROWS_PER_BLOCK, SIZE), lambda i: (i, 0)),
            pl.BlockSpec((LANES, N // LANES), lambda i: (0, 0)),
        ],
        out_specs=pl.BlockSpec(memory_space=pltpu.SMEM),
        out_shape=jax.ShapeDtypeStruct((1, 1), jnp.float32),
    )(x, g2)
    return out[0, 0]


def kernel(x, target):
    tgt = target.astype(jnp.int32)
    g = _sc_gather(x.reshape(N * SIZE), tgt)
    return _tc_loss(x, g)


# speedup vs baseline: 2.4519x; 1.0279x over previous
import math

import jax
import jax.numpy as jnp
from jax import lax
from jax.experimental import pallas as pl
from jax.experimental.pallas import tpu as pltpu
from jax.experimental.pallas import tpu_sc as plsc

N = 2048
SIZE = 32000
SMOOTHING = 0.1
EPS = SMOOTHING / (SIZE - 1)
CONF = 1.0 - SMOOTHING
C0 = (SIZE - 1) * EPS * math.log(EPS) + CONF * math.log(CONF)

NC = 2
NS = 16
LANES = 16
NW = NC * NS
BPW = N // NW

ROWS_PER_BLOCK = 32
NSTREAMS = 4
GRID = N // (ROWS_PER_BLOCK * NSTREAMS)


def _sc_gather_body(xflat, tgt, out, tgt_v, idx_v, val_v, sem):
    wid = lax.axis_index("s") * NC + lax.axis_index("c")
    base = wid * BPW
    pltpu.sync_copy(tgt.at[pl.ds(base, BPW)], tgt_v)
    for j in range(BPW // LANES):
        rows = (base + j * LANES) + lax.broadcasted_iota(jnp.int32, (LANES,), 0)
        idx_v[pl.ds(j * LANES, LANES)] = rows * SIZE + tgt_v[pl.ds(j * LANES, LANES)]
    pltpu.async_copy(xflat.at[idx_v], val_v, sem).wait()
    pltpu.sync_copy(val_v, out.at[pl.ds(base, BPW)])


def _sc_gather(xflat, tgt):
    k = pl.kernel(
        _sc_gather_body,
        out_type=jax.ShapeDtypeStruct((N,), jnp.float32),
        mesh=plsc.VectorSubcoreMesh(core_axis_name="c", subcore_axis_name="s"),
        scratch_types=[
            pltpu.VMEM((BPW,), jnp.int32),
            pltpu.VMEM((BPW,), jnp.int32),
            pltpu.VMEM((BPW,), jnp.float32),
            pltpu.SemaphoreType.DMA,
        ],
    )
    return k(xflat, tgt)


def _tc_loss_body(*refs):
    x_refs = refs[:NSTREAMS]
    g_ref = refs[NSTREAMS]
    out_ref = refs[NSTREAMS + 1]
    i = pl.program_id(0)

    @pl.when(i == 0)
    def _init():
        out_ref[0, 0] = jnp.float32(0.0)

    acc = jnp.float32(0.0)
    for r in x_refs:
        acc += jnp.sum(r[...])
    out_ref[0, 0] += acc

    @pl.when(i == GRID - 1)
    def _fin():
        s = out_ref[0, 0]
        g = jnp.sum(g_ref[...])
        out_ref[0, 0] = (
            jnp.float32(C0)
            - jnp.float32(EPS) * (s / N)
            + jnp.float32(EPS - CONF) * (g / N)
        )


def _tc_loss(x, gvals):
    g2 = gvals.reshape(LANES, N // LANES)
    x_specs = [
        pl.BlockSpec((ROWS_PER_BLOCK, SIZE), lambda i, k=k: (k * GRID + i, 0))
        for k in range(NSTREAMS)
    ]
    out = pl.pallas_call(
        _tc_loss_body,
        grid=(GRID,),
        in_specs=x_specs + [pl.BlockSpec((LANES, N // LANES), lambda i: (0, 0))],
        out_specs=pl.BlockSpec(memory_space=pltpu.SMEM),
        out_shape=jax.ShapeDtypeStruct((1, 1), jnp.float32),
    )(*([x] * NSTREAMS), g2)
    return out[0, 0]


def kernel(x, target):
    tgt = target.astype(jnp.int32)
    g = _sc_gather(x.reshape(N * SIZE), tgt)
    return _tc_loss(x, g)
